# baseline (speedup 1.0000x reference)
import functools

import jax
import jax.numpy as jnp
from jax import lax
from jax.experimental import pallas as pl
from jax.experimental.pallas import tpu as pltpu
from jax.experimental.pallas import tpu_sc as plsc

B, S, D = 4, 2048, 1024
NUM_STATES = 64
L = 16
NC, NS = 2, 16
NW = NC * NS
ROWS_PER_W = (B * S) // NW
R_CHUNK = 16
CH = R_CHUNK * D
N_CHUNKS = ROWS_PER_W // R_CHUNK


def _sc_kernel(h_hbm, idx_hbm, cv_hbm, o_hbm,
               idx_v, adj4, ib0, ib1, ob0, ob1,
               gsem, sin0, sin1, sout0, sout1):
    wid = lax.axis_index("s") * NC + lax.axis_index("c")
    base = wid * (ROWS_PER_W * D)
    b = wid // (NW // B)

    pltpu.sync_copy(idx_hbm, idx_v)
    pltpu.async_copy(cv_hbm.at[idx_v.at[pl.ds(0, B)]], adj4, gsem).wait()

    ibufs, obufs = (ib0, ib1), (ob0, ob1)
    sins, souts = (sin0, sin1), (sout0, sout1)

    def start_in(k):
        return pltpu.async_copy(
            h_hbm.at[pl.ds(base + k * CH, CH)], ibufs[k % 2], sins[k % 2])

    def start_out(k):
        return pltpu.async_copy(
            obufs[k % 2], o_hbm.at[pl.ds(base + k * CH, CH)], souts[k % 2])

    def compute(ib, ob):
        @pl.loop(0, D, step=L)
        def _(c1):
            a = adj4[b, pl.ds(c1, L)]
            for r in range(R_CHUNK):
                off = r * D
                ob[pl.ds(off + c1, L)] = ib[pl.ds(off + c1, L)] + a

    in_dma = [None] * N_CHUNKS
    out_dma = [None] * N_CHUNKS
    in_dma[0] = start_in(0)
    in_dma[1] = start_in(1)
    for k in range(N_CHUNKS):
        in_dma[k].wait()
        if k >= 2:
            out_dma[k - 2].wait()
        compute(ibufs[k % 2], obufs[k % 2])
        out_dma[k] = start_out(k)
        if k + 2 < N_CHUNKS:
            in_dma[k + 2] = start_in(k + 2)
    out_dma[N_CHUNKS - 2].wait()
    out_dma[N_CHUNKS - 1].wait()


@jax.jit
def kernel(hidden_states, affective_state_indices, control_vectors):
    h_flat = hidden_states.reshape(-1)
    idx = jnp.zeros((8,), jnp.int32).at[:B].set(
        affective_state_indices.astype(jnp.int32))

    mesh = plsc.VectorSubcoreMesh(core_axis_name="c", subcore_axis_name="s")
    run = pl.kernel(
        _sc_kernel,
        out_type=jax.ShapeDtypeStruct((B * S * D,), jnp.float32),
        mesh=mesh,
        scratch_types=[
            pltpu.VMEM((8,), jnp.int32),
            pltpu.VMEM((B, D), jnp.float32),
            pltpu.VMEM((CH,), jnp.float32),
            pltpu.VMEM((CH,), jnp.float32),
            pltpu.VMEM((CH,), jnp.float32),
            pltpu.VMEM((CH,), jnp.float32),
            pltpu.SemaphoreType.DMA,
            pltpu.SemaphoreType.DMA,
            pltpu.SemaphoreType.DMA,
            pltpu.SemaphoreType.DMA,
            pltpu.SemaphoreType.DMA,
        ],
    )
    out_flat = run(h_flat, idx, control_vectors)
    return out_flat.reshape(B, S, D)


# speedup vs baseline: 1.4748x; 1.4748x over previous
import jax
import jax.numpy as jnp
from jax import lax
from jax.experimental import pallas as pl
from jax.experimental.pallas import tpu as pltpu
from jax.experimental.pallas import tpu_sc as plsc

B, S, D = 4, 2048, 1024
NUM_STATES = 64
L = 16
NC, NS = 2, 16
NW = NC * NS
ROWS_PER_W = (B * S) // NW
R_CHUNK = 16
CH = R_CHUNK * D
N_CHUNKS = ROWS_PER_W // R_CHUNK
NBUF = 4


def _sc_kernel(h_hbm, idx_hbm, cv_hbm, o_hbm,
               idx_v, adj4, b0, b1, b2, b3,
               gsem, s0, s1, s2, s3):
    wid = lax.axis_index("s") * NC + lax.axis_index("c")
    base = wid * (ROWS_PER_W * D)
    b = wid // (NW // B)

    pltpu.sync_copy(idx_hbm, idx_v)
    pltpu.async_copy(cv_hbm.at[idx_v.at[pl.ds(0, B)]], adj4, gsem).wait()

    bufs = (b0, b1, b2, b3)
    sems = (s0, s1, s2, s3)

    def start_in(k):
        return pltpu.async_copy(
            h_hbm.at[pl.ds(base + k * CH, CH)], bufs[k % NBUF], sems[k % NBUF])

    def start_out(k):
        return pltpu.async_copy(
            bufs[k % NBUF], o_hbm.at[pl.ds(base + k * CH, CH)], sems[k % NBUF])

    def compute(buf):
        @plsc.parallel_loop(0, D, step=L, unroll=4)
        def _(c1):
            a = adj4[b, pl.ds(c1, L)]
            for r in range(R_CHUNK):
                plsc.addupdate(buf.at[pl.ds(r * D + c1, L)], a)

    in_dma = [None] * N_CHUNKS
    out_dma = [None] * N_CHUNKS
    in_dma[0] = start_in(0)
    in_dma[1] = start_in(1)
    for k in range(N_CHUNKS):
        in_dma[k].wait()
        compute(bufs[k % NBUF])
        out_dma[k] = start_out(k)
        if k >= 2:
            out_dma[k - 2].wait()
        if k + 2 < N_CHUNKS:
            in_dma[k + 2] = start_in(k + 2)
    out_dma[N_CHUNKS - 2].wait()
    out_dma[N_CHUNKS - 1].wait()


@jax.jit
def kernel(hidden_states, affective_state_indices, control_vectors):
    h_flat = hidden_states.reshape(-1)
    idx = jnp.zeros((8,), jnp.int32).at[:B].set(
        affective_state_indices.astype(jnp.int32))

    mesh = plsc.VectorSubcoreMesh(core_axis_name="c", subcore_axis_name="s")
    run = pl.kernel(
        _sc_kernel,
        out_type=jax.ShapeDtypeStruct((B * S * D,), jnp.float32),
        mesh=mesh,
        scratch_types=[
            pltpu.VMEM((8,), jnp.int32),
            pltpu.VMEM((B, D), jnp.float32),
            pltpu.VMEM((CH,), jnp.float32),
            pltpu.VMEM((CH,), jnp.float32),
            pltpu.VMEM((CH,), jnp.float32),
            pltpu.VMEM((CH,), jnp.float32),
            pltpu.SemaphoreType.DMA,
            pltpu.SemaphoreType.DMA,
            pltpu.SemaphoreType.DMA,
            pltpu.SemaphoreType.DMA,
            pltpu.SemaphoreType.DMA,
        ],
    )
    out_flat = run(h_flat, idx, control_vectors)
    return out_flat.reshape(B, S, D)


# speedup vs baseline: 2.5446x; 1.7254x over previous
import jax
import jax.numpy as jnp
from jax import lax
from jax.experimental import pallas as pl
from jax.experimental.pallas import tpu as pltpu
from jax.experimental.pallas import tpu_sc as plsc

B, S, D = 4, 2048, 1024
NUM_STATES = 64
ROWS = B * S
L = 16
NC, NS = 2, 16
NW = NC * NS
N_SC = 3072
ROWS_PER_W = N_SC // NW
R_CHUNK = 16
N_CHUNKS = ROWS_PER_W // R_CHUNK
NBUF = 4

TC_BLK = 512
N_TC_BLKS = (ROWS - N_SC) // TC_BLK


def _sc_kernel(h_hbm, idx_hbm, cv_hbm, o_hbm,
               idx_v, adj4, rows_v, b0, b1, b2, b3,
               gsem, s0, s1, s2, s3):
    wid = lax.axis_index("s") * NC + lax.axis_index("c")
    base_row = wid * ROWS_PER_W

    pltpu.sync_copy(idx_hbm, idx_v)
    pltpu.async_copy(cv_hbm.at[idx_v.at[pl.ds(0, B)]], adj4, gsem).wait()

    iota = lax.iota(jnp.int32, L)
    for k in range(N_CHUNKS):
        rows_v[k, :] = iota + (base_row + k * R_CHUNK)

    bufs = (b0, b1, b2, b3)
    sems = (s0, s1, s2, s3)

    def start_in(k):
        return pltpu.async_copy(
            h_hbm.at[rows_v.at[k]], bufs[k % NBUF], sems[k % NBUF])

    def start_out(k):
        return pltpu.async_copy(
            bufs[k % NBUF], o_hbm.at[rows_v.at[k]], sems[k % NBUF])

    def compute(k):
        buf = bufs[k % NBUF]
        b = (base_row + k * R_CHUNK) // S

        @plsc.parallel_loop(0, D, step=L, unroll=4)
        def _(c1):
            a = adj4[b, pl.ds(c1, L)]
            for r in range(R_CHUNK):
                plsc.addupdate(buf.at[r, pl.ds(c1, L)], a)

    in_dma = [None] * N_CHUNKS
    out_dma = [None] * N_CHUNKS
    in_dma[0] = start_in(0)
    in_dma[1] = start_in(1)
    for k in range(N_CHUNKS):
        in_dma[k].wait()
        compute(k)
        out_dma[k] = start_out(k)
        if k >= 2:
            out_dma[k - 2].wait()
        if k + 2 < N_CHUNKS:
            in_dma[k + 2] = start_in(k + 2)
    out_dma[N_CHUNKS - 2].wait()
    out_dma[N_CHUNKS - 1].wait()


def _sc_part(h2d, idx, cv):
    mesh = plsc.VectorSubcoreMesh(core_axis_name="c", subcore_axis_name="s")
    run = pl.kernel(
        _sc_kernel,
        out_type=jax.ShapeDtypeStruct((N_SC, D), jnp.float32),
        mesh=mesh,
        scratch_types=[
            pltpu.VMEM((8,), jnp.int32),
            pltpu.VMEM((B, D), jnp.float32),
            pltpu.VMEM((N_CHUNKS, L), jnp.int32),
            pltpu.VMEM((R_CHUNK, D), jnp.float32),
            pltpu.VMEM((R_CHUNK, D), jnp.float32),
            pltpu.VMEM((R_CHUNK, D), jnp.float32),
            pltpu.VMEM((R_CHUNK, D), jnp.float32),
            pltpu.SemaphoreType.DMA,
            pltpu.SemaphoreType.DMA,
            pltpu.SemaphoreType.DMA,
            pltpu.SemaphoreType.DMA,
            pltpu.SemaphoreType.DMA,
        ],
    )
    return run(h2d, idx, cv)


def _tc_add_kernel(idx_ref, h_ref, cv_ref, o_ref):
    o_ref[...] = h_ref[...] + cv_ref[0]


def _tc_part(h2d, idx, cv3):
    base_blk = N_SC // TC_BLK
    return pl.pallas_call(
        _tc_add_kernel,
        grid_spec=pltpu.PrefetchScalarGridSpec(
            num_scalar_prefetch=1,
            grid=(N_TC_BLKS,),
            in_specs=[
                pl.BlockSpec((TC_BLK, D), lambda i, idx_ref: (base_blk + i, 0)),
                pl.BlockSpec(
                    (1, 1, D),
                    lambda i, idx_ref: (
                        idx_ref[(base_blk + i) // (S // TC_BLK)], 0, 0)),
            ],
            out_specs=pl.BlockSpec((TC_BLK, D), lambda i, idx_ref: (i, 0)),
        ),
        out_shape=jax.ShapeDtypeStruct((ROWS - N_SC, D), jnp.float32),
    )(idx, h2d, cv3)


@jax.jit
def kernel(hidden_states, affective_state_indices, control_vectors):
    h2d = hidden_states.reshape(ROWS, D)
    idx32 = affective_state_indices.astype(jnp.int32)
    idx_pad = jnp.zeros((8,), jnp.int32).at[:B].set(idx32)
    cv3 = control_vectors.reshape(NUM_STATES, 1, D)

    out_sc = _sc_part(h2d, idx_pad, control_vectors)
    out_tc = _tc_part(h2d, idx32, cv3)
    return jnp.concatenate([out_sc, out_tc], axis=0).reshape(B, S, D)


# speedup vs baseline: 6.8120x; 2.6770x over previous
import jax
import jax.numpy as jnp
from jax.experimental import pallas as pl
from jax.experimental.pallas import tpu as pltpu

B, S, D = 4, 2048, 1024
NUM_STATES = 64
ROWS = B * S
R_BLK = 1024


def _add_kernel(idx_ref, h_ref, cv_ref, o_ref):
    o_ref[...] = h_ref[...] + cv_ref[0]


def kernel(hidden_states, affective_state_indices, control_vectors):
    idx = affective_state_indices.astype(jnp.int32)
    h2d = hidden_states.reshape(ROWS, D)
    cv3 = control_vectors.reshape(NUM_STATES, 1, D)
    blks_per_batch = S // R_BLK
    out = pl.pallas_call(
        _add_kernel,
        grid_spec=pltpu.PrefetchScalarGridSpec(
            num_scalar_prefetch=1,
            grid=(ROWS // R_BLK,),
            in_specs=[
                pl.BlockSpec((R_BLK, D), lambda i, idx_ref: (i, 0)),
                pl.BlockSpec(
                    (1, 1, D),
                    lambda i, idx_ref: (idx_ref[i // blks_per_batch], 0, 0)),
            ],
            out_specs=pl.BlockSpec((R_BLK, D), lambda i, idx_ref: (i, 0)),
        ),
        out_shape=jax.ShapeDtypeStruct((ROWS, D), jnp.float32),
    )(idx, h2d, cv3)
    return out.reshape(B, S, D)


# speedup vs baseline: 7.2709x; 1.0674x over previous
import jax
import jax.numpy as jnp
from jax.experimental import pallas as pl
from jax.experimental.pallas import tpu as pltpu

B, S, D = 4, 2048, 1024
NUM_STATES = 64
ROWS = B * S
R_BLK = 2048


def _add_kernel(idx_ref, h_ref, cv_ref, o_ref):
    o_ref[...] = h_ref[...] + cv_ref[0]


def kernel(hidden_states, affective_state_indices, control_vectors):
    idx = affective_state_indices.astype(jnp.int32)
    h2d = hidden_states.reshape(ROWS, D)
    cv3 = control_vectors.reshape(NUM_STATES, 1, D)
    blks_per_batch = S // R_BLK
    out = pl.pallas_call(
        _add_kernel,
        grid_spec=pltpu.PrefetchScalarGridSpec(
            num_scalar_prefetch=1,
            grid=(ROWS // R_BLK,),
            in_specs=[
                pl.BlockSpec((R_BLK, D), lambda i, idx_ref: (i, 0)),
                pl.BlockSpec(
                    (1, 1, D),
                    lambda i, idx_ref: (idx_ref[i // blks_per_batch], 0, 0)),
            ],
            out_specs=pl.BlockSpec((R_BLK, D), lambda i, idx_ref: (i, 0)),
        ),
        out_shape=jax.ShapeDtypeStruct((ROWS, D), jnp.float32),
    )(idx, h2d, cv3)
    return out.reshape(B, S, D)
